# Initial kernel scaffold; baseline (speedup 1.0000x reference)
import jax
import jax.numpy as jnp
from jax.experimental import pallas as pl


def kernel(z, edge_index, edge_type, rel_emb, rel_emb_imag):
    raise NotImplementedError("write your pallas kernel here")


# speedup vs baseline: 5.6208x; 5.6208x over previous
import functools

import jax
import jax.numpy as jnp
from jax import lax
from jax.experimental import pallas as pl
from jax.experimental.pallas import tpu as pltpu
from jax.experimental.pallas import tpu_sc as plsc

NUM_NODES = 10000
NUM_EDGES = 320000
NUM_REL = 500
H = 64
ZD = 2 * H
NC = 2
NS = 16
L = 16
NW = NC * NS
EPW = NUM_EDGES // NW
CHUNK = 80
NCHUNK = EPW // CHUNK
GROUPS = CHUNK // L


def _score_body(z_hbm, hidx_hbm, tidx_hbm, et_hbm, rel_hbm, reli_hbm,
                out_hbm,
                rel_v, reli_v, head_v, tail_v, hidx_v, tidx_v, et_v,
                scores_v, sem_h, sem_t):
    wid = lax.axis_index("s") * NC + lax.axis_index("c")
    base0 = wid * EPW
    pltpu.sync_copy(rel_hbm, rel_v)
    pltpu.sync_copy(reli_hbm, reli_v)
    lane = lax.iota(jnp.int32, L)

    def chunk_body(i, _):
        base = base0 + i * CHUNK
        pltpu.sync_copy(hidx_hbm.at[pl.ds(base, CHUNK)], hidx_v)
        pltpu.sync_copy(tidx_hbm.at[pl.ds(base, CHUNK)], tidx_v)
        pltpu.sync_copy(et_hbm.at[pl.ds(base, CHUNK)], et_v)
        hc = pltpu.async_copy(z_hbm.at[hidx_v], head_v, sem_h)
        tc = pltpu.async_copy(z_hbm.at[tidx_v], tail_v, sem_t)
        hc.wait()
        tc.wait()

        def group_body(g, _):
            rows = g * L + lane
            et = et_v[pl.ds(g * L, L)]
            etoff = et * H

            def d_body(d, acc):
                col = (d + lane) & (H - 1)
                coli = col + H
                hr = plsc.load_gather(head_v, [rows, col])
                hi = plsc.load_gather(head_v, [rows, coli])
                tr = plsc.load_gather(tail_v, [rows, col])
                ti = plsc.load_gather(tail_v, [rows, coli])
                raddr = etoff + col
                rr = plsc.load_gather(rel_v, [raddr])
                ri = plsc.load_gather(reli_v, [raddr])
                return acc + (hr * rr - hi * ri) * tr + (hr * ri + hi * rr) * ti

            acc = lax.fori_loop(0, H, d_body, jnp.zeros((L,), jnp.float32))
            scores_v[pl.ds(i * CHUNK + g * L, L)] = acc
            return 0

        lax.fori_loop(0, GROUPS, group_body, 0)
        return 0

    lax.fori_loop(0, NCHUNK, chunk_body, 0)
    pltpu.sync_copy(scores_v, out_hbm.at[pl.ds(base0, EPW)])


def kernel(z, edge_index, edge_type, rel_emb, rel_emb_imag):
    hidx = edge_index[0].astype(jnp.int32)
    tidx = edge_index[1].astype(jnp.int32)
    et = edge_type.astype(jnp.int32)
    mesh = plsc.VectorSubcoreMesh(
        core_axis_name="c", subcore_axis_name="s", num_cores=NC, num_subcores=NS
    )
    run = pl.kernel(
        _score_body,
        out_type=jax.ShapeDtypeStruct((NUM_EDGES,), jnp.float32),
        mesh=mesh,
        compiler_params=pltpu.CompilerParams(needs_layout_passes=False),
        scratch_types=[
            pltpu.VMEM((NUM_REL * H,), jnp.float32),
            pltpu.VMEM((NUM_REL * H,), jnp.float32),
            pltpu.VMEM((CHUNK, ZD), jnp.float32),
            pltpu.VMEM((CHUNK, ZD), jnp.float32),
            pltpu.VMEM((CHUNK,), jnp.int32),
            pltpu.VMEM((CHUNK,), jnp.int32),
            pltpu.VMEM((CHUNK,), jnp.int32),
            pltpu.VMEM((EPW,), jnp.float32),
            pltpu.SemaphoreType.DMA,
            pltpu.SemaphoreType.DMA,
        ],
    )
    return run(z, hidx, tidx, et, rel_emb.reshape(-1), rel_emb_imag.reshape(-1))


# speedup vs baseline: 6.2903x; 1.1191x over previous
import functools

import jax
import jax.numpy as jnp
from jax import lax
from jax.experimental import pallas as pl
from jax.experimental.pallas import tpu as pltpu
from jax.experimental.pallas import tpu_sc as plsc

NUM_NODES = 10000
NUM_EDGES = 320000
NUM_REL = 500
H = 64
ZD = 2 * H
NC = 2
NS = 16
L = 16
NW = NC * NS
EPW = NUM_EDGES // NW
CHUNK = 80
NCHUNK = EPW // CHUNK
GROUPS = CHUNK // L


def _score_body(z_hbm, hidx_hbm, tidx_hbm, et_hbm, rel_hbm, reli_hbm,
                out_hbm,
                rel_v, reli_v, head_v, tail_v, hidx_v, tidx_v, et_v,
                scores_v, sem_h, sem_t):
    wid = lax.axis_index("s") * NC + lax.axis_index("c")
    base0 = wid * EPW
    pltpu.sync_copy(rel_hbm, rel_v)
    pltpu.sync_copy(reli_hbm, reli_v)
    lane = lax.iota(jnp.int32, L)

    def chunk_body(i, _):
        base = base0 + i * CHUNK
        pltpu.sync_copy(hidx_hbm.at[pl.ds(base, CHUNK)], hidx_v)
        pltpu.sync_copy(tidx_hbm.at[pl.ds(base, CHUNK)], tidx_v)
        pltpu.sync_copy(et_hbm.at[pl.ds(base, CHUNK)], et_v)
        hc = pltpu.async_copy(z_hbm.at[hidx_v], head_v, sem_h)
        tc = pltpu.async_copy(z_hbm.at[tidx_v], tail_v, sem_t)
        hc.wait()
        tc.wait()

        for g in range(GROUPS):
            rows = g * L + lane
            et = et_v[pl.ds(g * L, L)]
            etoff = et * H

            def d_body(d, acc, rows=rows, etoff=etoff):
                col = (d + lane) & (H - 1)
                coli = col + H
                hr = plsc.load_gather(head_v, [rows, col])
                hi = plsc.load_gather(head_v, [rows, coli])
                tr = plsc.load_gather(tail_v, [rows, col])
                ti = plsc.load_gather(tail_v, [rows, coli])
                raddr = etoff + col
                rr = plsc.load_gather(rel_v, [raddr])
                ri = plsc.load_gather(reli_v, [raddr])
                return acc + (hr * rr - hi * ri) * tr + (hr * ri + hi * rr) * ti

            acc = lax.fori_loop(0, H, d_body, jnp.zeros((L,), jnp.float32),
                                unroll=8)
            scores_v[pl.ds(i * CHUNK + g * L, L)] = acc
        return 0

    lax.fori_loop(0, NCHUNK, chunk_body, 0)
    pltpu.sync_copy(scores_v, out_hbm.at[pl.ds(base0, EPW)])


def kernel(z, edge_index, edge_type, rel_emb, rel_emb_imag):
    hidx = edge_index[0].astype(jnp.int32)
    tidx = edge_index[1].astype(jnp.int32)
    et = edge_type.astype(jnp.int32)
    mesh = plsc.VectorSubcoreMesh(
        core_axis_name="c", subcore_axis_name="s", num_cores=NC, num_subcores=NS
    )
    run = pl.kernel(
        _score_body,
        out_type=jax.ShapeDtypeStruct((NUM_EDGES,), jnp.float32),
        mesh=mesh,
        compiler_params=pltpu.CompilerParams(needs_layout_passes=False),
        scratch_types=[
            pltpu.VMEM((NUM_REL * H,), jnp.float32),
            pltpu.VMEM((NUM_REL * H,), jnp.float32),
            pltpu.VMEM((CHUNK, ZD), jnp.float32),
            pltpu.VMEM((CHUNK, ZD), jnp.float32),
            pltpu.VMEM((CHUNK,), jnp.int32),
            pltpu.VMEM((CHUNK,), jnp.int32),
            pltpu.VMEM((CHUNK,), jnp.int32),
            pltpu.VMEM((EPW,), jnp.float32),
            pltpu.SemaphoreType.DMA,
            pltpu.SemaphoreType.DMA,
        ],
    )
    return run(z, hidx, tidx, et, rel_emb.reshape(-1), rel_emb_imag.reshape(-1))


# speedup vs baseline: 12.6394x; 2.0093x over previous
import functools

import jax
import jax.numpy as jnp
from jax import lax
from jax.experimental import pallas as pl
from jax.experimental.pallas import tpu as pltpu
from jax.experimental.pallas import tpu_sc as plsc

NUM_NODES = 10000
NUM_EDGES = 320000
NUM_REL = 500
H = 64
ZD = 2 * H
NC = 2
NS = 16
L = 16
NW = NC * NS
EPW = NUM_EDGES // NW
CHUNK = 80
NCHUNK = EPW // CHUNK
GROUPS = CHUNK // L


def _score_body(z_hbm, hidx_hbm, tidx_hbm, et_hbm, rel_hbm, reli_hbm,
                out_hbm,
                rel_v, reli_v, hidx_all, tidx_all,
                head_v0, head_v1, tail_v0, tail_v1,
                et_v0, et_v1, sc_v0, sc_v1,
                sem_g0, sem_g1, sem_e0, sem_e1, sem_o0, sem_o1):
    wid = lax.axis_index("s") * NC + lax.axis_index("c")
    base0 = wid * EPW
    pltpu.sync_copy(rel_hbm, rel_v)
    pltpu.sync_copy(reli_hbm, reli_v)
    pltpu.sync_copy(hidx_hbm.at[pl.ds(base0, EPW)], hidx_all)
    pltpu.sync_copy(tidx_hbm.at[pl.ds(base0, EPW)], tidx_all)
    lane = lax.iota(jnp.int32, L)

    head_v = (head_v0, head_v1)
    tail_v = (tail_v0, tail_v1)
    et_v = (et_v0, et_v1)
    sc_v = (sc_v0, sc_v1)
    sem_g = (sem_g0, sem_g1)
    sem_e = (sem_e0, sem_e1)
    sem_o = (sem_o0, sem_o1)

    def io(i, b):
        off = i * CHUNK
        pltpu.make_async_copy(
            et_hbm.at[pl.ds(base0 + off, CHUNK)], et_v[b], sem_e[b]).start()
        pltpu.make_async_copy(
            z_hbm.at[hidx_all.at[pl.ds(off, CHUNK)]], head_v[b], sem_g[b]).start()
        pltpu.make_async_copy(
            z_hbm.at[tidx_all.at[pl.ds(off, CHUNK)]], tail_v[b], sem_g[b]).start()

    def compute(i, b):
        pltpu.make_async_copy(
            et_hbm.at[pl.ds(base0, CHUNK)], et_v[b], sem_e[b]).wait()
        pltpu.make_async_copy(
            z_hbm.at[hidx_all.at[pl.ds(0, CHUNK)]], head_v[b], sem_g[b]).wait()
        pltpu.make_async_copy(
            z_hbm.at[hidx_all.at[pl.ds(0, CHUNK)]], tail_v[b], sem_g[b]).wait()

        @pl.when(i >= 2)
        def _():
            pltpu.make_async_copy(
                sc_v[b], out_hbm.at[pl.ds(base0, CHUNK)], sem_o[b]).wait()

        for g in range(GROUPS):
            rows = g * L + lane
            et = et_v[b][pl.ds(g * L, L)]
            etoff = et * H

            def d_body(d, acc, rows=rows, etoff=etoff):
                col = (d + lane) & (H - 1)
                coli = col + H
                hr = plsc.load_gather(head_v[b], [rows, col])
                hi = plsc.load_gather(head_v[b], [rows, coli])
                tr = plsc.load_gather(tail_v[b], [rows, col])
                ti = plsc.load_gather(tail_v[b], [rows, coli])
                raddr = etoff + col
                rr = plsc.load_gather(rel_v, [raddr])
                ri = plsc.load_gather(reli_v, [raddr])
                return acc + (hr * rr - hi * ri) * tr + (hr * ri + hi * rr) * ti

            acc = lax.fori_loop(0, H, d_body, jnp.zeros((L,), jnp.float32),
                                unroll=8)
            sc_v[b][pl.ds(g * L, L)] = acc

        pltpu.make_async_copy(
            sc_v[b], out_hbm.at[pl.ds(base0 + i * CHUNK, CHUNK)], sem_o[b]).start()

    io(0, 0)

    def pair_body(p, _):
        i = 1 + 2 * p
        io(i, 1)
        compute(i - 1, 0)
        io(i + 1, 0)
        compute(i, 1)
        return 0

    lax.fori_loop(0, (NCHUNK - 1) // 2, pair_body, 0)
    compute(NCHUNK - 1, 0)
    pltpu.make_async_copy(sc_v1, out_hbm.at[pl.ds(base0, CHUNK)], sem_o1).wait()
    pltpu.make_async_copy(sc_v0, out_hbm.at[pl.ds(base0, CHUNK)], sem_o0).wait()


def kernel(z, edge_index, edge_type, rel_emb, rel_emb_imag):
    hidx = edge_index[0].astype(jnp.int32)
    tidx = edge_index[1].astype(jnp.int32)
    et = edge_type.astype(jnp.int32)
    mesh = plsc.VectorSubcoreMesh(
        core_axis_name="c", subcore_axis_name="s", num_cores=NC, num_subcores=NS
    )
    run = pl.kernel(
        _score_body,
        out_type=jax.ShapeDtypeStruct((NUM_EDGES,), jnp.float32),
        mesh=mesh,
        compiler_params=pltpu.CompilerParams(needs_layout_passes=False),
        scratch_types=[
            pltpu.VMEM((NUM_REL * H,), jnp.float32),
            pltpu.VMEM((NUM_REL * H,), jnp.float32),
            pltpu.VMEM((EPW,), jnp.int32),
            pltpu.VMEM((EPW,), jnp.int32),
            pltpu.VMEM((CHUNK, ZD), jnp.float32),
            pltpu.VMEM((CHUNK, ZD), jnp.float32),
            pltpu.VMEM((CHUNK, ZD), jnp.float32),
            pltpu.VMEM((CHUNK, ZD), jnp.float32),
            pltpu.VMEM((CHUNK,), jnp.int32),
            pltpu.VMEM((CHUNK,), jnp.int32),
            pltpu.VMEM((CHUNK,), jnp.float32),
            pltpu.VMEM((CHUNK,), jnp.float32),
            pltpu.SemaphoreType.DMA,
            pltpu.SemaphoreType.DMA,
            pltpu.SemaphoreType.DMA,
            pltpu.SemaphoreType.DMA,
            pltpu.SemaphoreType.DMA,
            pltpu.SemaphoreType.DMA,
        ],
    )
    return run(z, hidx, tidx, et, rel_emb.reshape(-1), rel_emb_imag.reshape(-1))


# speedup vs baseline: 12.7424x; 1.0081x over previous
import functools

import jax
import jax.numpy as jnp
from jax import lax
from jax.experimental import pallas as pl
from jax.experimental.pallas import tpu as pltpu
from jax.experimental.pallas import tpu_sc as plsc

NUM_NODES = 10000
NUM_EDGES = 320000
NUM_REL = 500
H = 64
ZD = 2 * H
NC = 2
NS = 16
L = 16
NW = NC * NS
EPW = NUM_EDGES // NW
CHUNK = 80
NCHUNK = EPW // CHUNK
GROUPS = CHUNK // L


def _score_body(z_hbm, hidx_hbm, tidx_hbm, et_hbm, rel_hbm, reli_hbm,
                out_hbm,
                rel_v, reli_v, hidx_all, tidx_all,
                head_v0, head_v1, tail_v0, tail_v1,
                et_v0, et_v1, sc_v0, sc_v1,
                sem_g0, sem_g1, sem_e0, sem_e1, sem_o0, sem_o1):
    sid = lax.axis_index("s")
    wid = sid * NC + lax.axis_index("c")
    base0 = wid * EPW

    pltpu.sync_copy(rel_hbm, rel_v)
    pltpu.sync_copy(reli_hbm, reli_v)
    pltpu.sync_copy(hidx_hbm.at[pl.ds(base0, EPW)], hidx_all)
    pltpu.sync_copy(tidx_hbm.at[pl.ds(base0, EPW)], tidx_all)
    lane = lax.iota(jnp.int32, L)

    head_v = (head_v0, head_v1)
    tail_v = (tail_v0, tail_v1)
    et_v = (et_v0, et_v1)
    sc_v = (sc_v0, sc_v1)
    sem_g = (sem_g0, sem_g1)
    sem_e = (sem_e0, sem_e1)
    sem_o = (sem_o0, sem_o1)

    def io(i, b):
        off = i * CHUNK
        pltpu.make_async_copy(
            et_hbm.at[pl.ds(base0 + off, CHUNK)], et_v[b], sem_e[b]).start()
        pltpu.make_async_copy(
            z_hbm.at[hidx_all.at[pl.ds(off, CHUNK)]], head_v[b], sem_g[b]).start()
        pltpu.make_async_copy(
            z_hbm.at[tidx_all.at[pl.ds(off, CHUNK)]], tail_v[b], sem_g[b]).start()

    def compute(i, b):
        pltpu.make_async_copy(
            et_hbm.at[pl.ds(base0, CHUNK)], et_v[b], sem_e[b]).wait()
        pltpu.make_async_copy(
            z_hbm.at[hidx_all.at[pl.ds(0, CHUNK)]], head_v[b], sem_g[b]).wait()
        pltpu.make_async_copy(
            z_hbm.at[hidx_all.at[pl.ds(0, CHUNK)]], tail_v[b], sem_g[b]).wait()

        @pl.when(i >= 2)
        def _():
            pltpu.make_async_copy(
                sc_v[b], out_hbm.at[pl.ds(base0, CHUNK)], sem_o[b]).wait()

        for g in range(GROUPS):
            rows = g * L + lane
            et = et_v[b][pl.ds(g * L, L)]
            etoff = et * H

            def d_body(d, acc, rows=rows, etoff=etoff):
                col = (d + 2 * lane) & (H - 1)
                coli = col + H
                hr = plsc.load_gather(head_v[b], [rows, col])
                hi = plsc.load_gather(head_v[b], [rows, coli])
                tr = plsc.load_gather(tail_v[b], [rows, col])
                ti = plsc.load_gather(tail_v[b], [rows, coli])
                raddr = etoff + col
                rr = plsc.load_gather(rel_v, [raddr])
                ri = plsc.load_gather(reli_v, [raddr])
                return acc + (hr * rr - hi * ri) * tr + (hr * ri + hi * rr) * ti

            acc = lax.fori_loop(0, H, d_body, jnp.zeros((L,), jnp.float32),
                                unroll=8)
            sc_v[b][pl.ds(g * L, L)] = acc

        pltpu.make_async_copy(
            sc_v[b], out_hbm.at[pl.ds(base0 + i * CHUNK, CHUNK)], sem_o[b]).start()

    io(0, 0)

    def pair_body(p, _):
        i = 1 + 2 * p
        io(i, 1)
        compute(i - 1, 0)
        io(i + 1, 0)
        compute(i, 1)
        return 0

    lax.fori_loop(0, (NCHUNK - 1) // 2, pair_body, 0)
    compute(NCHUNK - 1, 0)
    pltpu.make_async_copy(sc_v1, out_hbm.at[pl.ds(base0, CHUNK)], sem_o1).wait()
    pltpu.make_async_copy(sc_v0, out_hbm.at[pl.ds(base0, CHUNK)], sem_o0).wait()


def kernel(z, edge_index, edge_type, rel_emb, rel_emb_imag):
    hidx = edge_index[0].astype(jnp.int32)
    tidx = edge_index[1].astype(jnp.int32)
    et = edge_type.astype(jnp.int32)
    mesh = plsc.VectorSubcoreMesh(
        core_axis_name="c", subcore_axis_name="s", num_cores=NC, num_subcores=NS
    )
    run = pl.kernel(
        _score_body,
        out_type=jax.ShapeDtypeStruct((NUM_EDGES,), jnp.float32),
        mesh=mesh,
        compiler_params=pltpu.CompilerParams(needs_layout_passes=False),
        scratch_types=[
            pltpu.VMEM((NUM_REL * H,), jnp.float32),
            pltpu.VMEM((NUM_REL * H,), jnp.float32),
            pltpu.VMEM((EPW,), jnp.int32),
            pltpu.VMEM((EPW,), jnp.int32),
            pltpu.VMEM((CHUNK, ZD), jnp.float32),
            pltpu.VMEM((CHUNK, ZD), jnp.float32),
            pltpu.VMEM((CHUNK, ZD), jnp.float32),
            pltpu.VMEM((CHUNK, ZD), jnp.float32),
            pltpu.VMEM((CHUNK,), jnp.int32),
            pltpu.VMEM((CHUNK,), jnp.int32),
            pltpu.VMEM((CHUNK,), jnp.float32),
            pltpu.VMEM((CHUNK,), jnp.float32),
            pltpu.SemaphoreType.DMA,
            pltpu.SemaphoreType.DMA,
            pltpu.SemaphoreType.DMA,
            pltpu.SemaphoreType.DMA,
            pltpu.SemaphoreType.DMA,
            pltpu.SemaphoreType.DMA,
        ],
    )
    return run(z, hidx, tidx, et, rel_emb.reshape(-1), rel_emb_imag.reshape(-1))
